# Initial kernel scaffold; baseline (speedup 1.0000x reference)
import jax
import jax.numpy as jnp
from jax.experimental import pallas as pl


def kernel(x, edge_index, lin_x_W, lin_x_b, W0, b0, W1, b1, W2, b2):
    raise NotImplementedError("write your pallas kernel here")


# speedup vs baseline: 3.4709x; 3.4709x over previous
import functools

import jax
import jax.numpy as jnp
from jax import lax
from jax.experimental import pallas as pl
from jax.experimental.pallas import tpu as pltpu
from jax.experimental.pallas import tpu_sc as plsc

N = 10000
E = 320000
GR = 64
D = 128
NEG = -3.0e38
CHUNK = 6400
L = 16


def _take(v, idx):
  return jnp.take_along_axis(v, idx, axis=0, mode="promise_in_bounds")


def _segmax_body(pq_hbm, src_hbm, dst_hbm, out_hbm, q0, q1, a0, a1, es, ed):
  c = lax.axis_index("c")
  s = lax.axis_index("s")
  w = s * 2 + c
  f0 = 2 * w

  pltpu.sync_copy(pq_hbm.at[GR + f0], q0)
  pltpu.sync_copy(pq_hbm.at[GR + f0 + 1], q1)

  neg = jnp.full((L,), NEG, jnp.float32)

  def init(i, carry):
    a0[pl.ds(i * L, L)] = neg
    a1[pl.ds(i * L, L)] = neg
    return carry

  lax.fori_loop(0, N // L, init, 0)

  iota = lax.iota(jnp.int32, L)
  shift_idx = [jnp.maximum(iota - sh, 0) for sh in (1, 2, 4, 8)]
  nxt_idx = jnp.minimum(iota + 1, L - 1)
  last_lane = iota == L - 1

  def chunk_body(ci, carry):
    pltpu.sync_copy(src_hbm.at[pl.ds(ci * CHUNK, CHUNK)], es)
    pltpu.sync_copy(dst_hbm.at[pl.ds(ci * CHUNK, CHUNK)], ed)

    def vec_body(k, carry2):
      sv = es[pl.ds(k * L, L)]
      dv = ed[pl.ds(k * L, L)]
      dk, perm = plsc.sort_key_val(dv, iota)
      sp = _take(sv, perm)
      v0 = plsc.load_gather(q0, [sp])
      v1 = plsc.load_gather(q1, [sp])
      for ix in shift_idx:
        kk = _take(dk, ix)
        same = dk == kk
        v0 = jnp.where(same, jnp.maximum(v0, _take(v0, ix)), v0)
        v1 = jnp.where(same, jnp.maximum(v1, _take(v1, ix)), v1)
      is_last = (dk != _take(dk, nxt_idx)) | last_lane
      c0 = plsc.load_gather(a0, [dk])
      c1 = plsc.load_gather(a1, [dk])
      plsc.store_scatter(a0, [dk], jnp.maximum(c0, v0), mask=is_last)
      plsc.store_scatter(a1, [dk], jnp.maximum(c1, v1), mask=is_last)
      return carry2

    lax.fori_loop(0, CHUNK // L, vec_body, 0)
    return carry

  lax.fori_loop(0, E // CHUNK, chunk_body, 0)

  pltpu.sync_copy(a0, out_hbm.at[f0])
  pltpu.sync_copy(a1, out_hbm.at[f0 + 1])


_segmax = functools.partial(
    pl.kernel,
    mesh=plsc.VectorSubcoreMesh(core_axis_name="c", subcore_axis_name="s"),
    out_type=jax.ShapeDtypeStruct((GR, N), jnp.float32),
    scratch_types=[
        pltpu.VMEM((N,), jnp.float32),
        pltpu.VMEM((N,), jnp.float32),
        pltpu.VMEM((N,), jnp.float32),
        pltpu.VMEM((N,), jnp.float32),
        pltpu.VMEM((CHUNK,), jnp.int32),
        pltpu.VMEM((CHUNK,), jnp.int32),
    ],
    compiler_params=pltpu.CompilerParams(needs_layout_passes=False),
)(_segmax_body)


def _tc0_body(xT, WlT, bl, Wc, bc, h0T_o, pq_o):
  h0 = jnp.dot(WlT[...], xT[...], preferred_element_type=jnp.float32) + bl[...]
  h0T_o[...] = h0
  pq_o[...] = jnp.dot(Wc[...], h0, preferred_element_type=jnp.float32) + bc[...]


_tc0 = pl.pallas_call(
    _tc0_body,
    out_shape=[
        jax.ShapeDtypeStruct((GR, N), jnp.float32),
        jax.ShapeDtypeStruct((2 * GR, N), jnp.float32),
    ],
)


def _tcb_body(nparts, pq, mT, Wc, bc, *refs):
  hrefs = refs[:nparts]
  agg_o, pq_o = refs[nparts], refs[nparts + 1]
  m = mT[...]
  agg = jnp.where(m > -1.0e30, pq[0:GR, :] + m, 0.0)
  agg_o[...] = agg
  hcat = jnp.concatenate([h[...] for h in hrefs] + [agg], axis=0)
  pq_o[...] = jnp.dot(Wc[...], hcat, preferred_element_type=jnp.float32) + bc[...]


def _make_tcb(nparts):
  return pl.pallas_call(
      functools.partial(_tcb_body, nparts),
      out_shape=[
          jax.ShapeDtypeStruct((GR, N), jnp.float32),
          jax.ShapeDtypeStruct((2 * GR, N), jnp.float32),
      ],
  )


_tcb1 = _make_tcb(1)
_tcb2 = _make_tcb(2)


def _pool4(S):
  return jnp.max(S.reshape(GR // 4, 4, S.shape[-1]), axis=1)


def _tcf_body(h0T, a0T, a1T, pq, mT, out_o):
  m = mT[...]
  a2 = jnp.where(m > -1.0e30, pq[0:GR, :] + m, 0.0)
  out_o[...] = jnp.concatenate(
      [_pool4(h0T[...]), _pool4(a0T[...]), _pool4(a1T[...]), _pool4(a2)],
      axis=0,
  )


_tcf = pl.pallas_call(
    _tcf_body,
    out_shape=jax.ShapeDtypeStruct((GR, N), jnp.float32),
)


def kernel(x, edge_index, lin_x_W, lin_x_b, W0, b0, W1, b1, W2, b2):
  xT = x.T
  src = edge_index[0]
  dst = edge_index[1]

  Wcs, bcs = [], []
  for i, (W, b) in enumerate(((W0, b0), (W1, b1), (W2, b2))):
    cin = (i + 1) * GR
    Wt = W[:cin].T
    Wb = W[cin:].T
    Wcs.append(jnp.concatenate([Wt - Wb, Wb], axis=0))
    bcs.append(jnp.concatenate([b, jnp.zeros((GR,), jnp.float32)])[:, None])

  h0T, pq = _tc0(xT, lin_x_W.T, lin_x_b[:, None], Wcs[0], bcs[0])
  m0 = _segmax(pq, src, dst)
  agg0, pq = _tcb1(pq, m0, Wcs[1], bcs[1], h0T)
  m1 = _segmax(pq, src, dst)
  agg1, pq = _tcb2(pq, m1, Wcs[2], bcs[2], h0T, agg0)
  m2 = _segmax(pq, src, dst)
  outT = _tcf(h0T, agg0, agg1, pq, m2)
  return outT.T
